# baseline (speedup 1.0000x reference)
import functools

import jax
import jax.numpy as jnp
import numpy as np
from jax.experimental import pallas as pl
from jax.experimental.pallas import tpu as pltpu

VOCAB = 1000
PV = 1024
NTOK = 65536
TB = 256
NSTEPS = NTOK // TB

_TINY = np.float32(np.finfo(np.float32).tiny)
_K0 = 0
_K1 = 42
_KS2 = np.uint32(_K0 ^ _K1 ^ 0x1BD11BDA)
_ROT = ((13, 15, 26, 6), (17, 29, 16, 24))


def _rotl(x, r):
    return (x << jnp.uint32(r)) | (x >> jnp.uint32(32 - r))


def _threefry_bits(cnt_lo):
    ks = (jnp.uint32(_K0), jnp.uint32(_K1), _KS2)
    x0 = jnp.full_like(cnt_lo, ks[0])
    x1 = cnt_lo + ks[1]
    for i in range(5):
        for r in _ROT[i % 2]:
            x0 = x0 + x1
            x1 = _rotl(x1, r)
            x1 = x1 ^ x0
        x0 = x0 + ks[(i + 1) % 3]
        x1 = x1 + ks[(i + 2) % 3] + jnp.uint32(i + 1)
    return x0 ^ x1


def _body(mb_ref, y_ref, tab_ref, yp_ref, loss_ref, lse_ref, acc_ref):
    step = pl.program_id(0)
    col = jax.lax.broadcasted_iota(jnp.int32, (TB, PV), 1)
    valid = col < VOCAB

    @pl.when(step == 0)
    def _init():
        t = tab_ref[...]
        vcol = jax.lax.broadcasted_iota(jnp.int32, (PV, PV), 1)
        vvalid = vcol < VOCAB
        m = jnp.max(jnp.where(vvalid, t, -jnp.inf), axis=1, keepdims=True)
        s = jnp.sum(jnp.where(vvalid, jnp.exp(t - m), 0.0), axis=1,
                    keepdims=True)
        lse_ref[...] = m + jnp.log(s)
        acc_ref[0, 0] = 0.0

    mb = mb_ref[0]
    yb = y_ref[0]

    onehot = (col == mb).astype(jnp.float32)
    logits = jnp.dot(onehot, tab_ref[...],
                     preferred_element_type=jnp.float32,
                     precision=jax.lax.Precision.HIGHEST)

    row = jax.lax.broadcasted_iota(jnp.int32, (TB, PV), 0)
    cnt = (step * (TB * VOCAB) + row * VOCAB + col).astype(jnp.uint32)
    bits = _threefry_bits(cnt)

    fb = (bits >> jnp.uint32(9)) | jnp.uint32(0x3F800000)
    f = jax.lax.bitcast_convert_type(fb, jnp.float32) - jnp.float32(1.0)
    u = jnp.maximum(_TINY, f * (jnp.float32(1.0) - _TINY) + _TINY)
    g = -jnp.log(-jnp.log(u))

    z = jnp.where(valid, logits + g, -jnp.inf)
    zmax = jnp.max(z, axis=1, keepdims=True)
    win = jnp.min(jnp.where(z == zmax, col, PV), axis=1, keepdims=True)
    yp_ref[0] = win

    tgt = jnp.sum(jnp.where(col == yb, logits, 0.0), axis=1, keepdims=True)
    lse_tok = jnp.dot(onehot, lse_ref[...],
                      preferred_element_type=jnp.float32,
                      precision=jax.lax.Precision.HIGHEST)
    acc_ref[0, 0] += jnp.sum(lse_tok - tgt)
    loss_ref[...] = jnp.full((1, 1), acc_ref[0, 0] * jnp.float32(1.0 / NTOK),
                             jnp.float32)


@jax.jit
def kernel(mini_batch, y, embed_weight):
    mb = mini_batch.reshape(NSTEPS, TB, 1)
    yv = y.reshape(NSTEPS, TB, 1)
    tab = jnp.pad(embed_weight,
                  ((0, PV - VOCAB), (0, PV - VOCAB)))

    yp, loss = pl.pallas_call(
        _body,
        grid=(NSTEPS,),
        in_specs=[
            pl.BlockSpec((1, TB, 1), lambda i: (i, 0, 0)),
            pl.BlockSpec((1, TB, 1), lambda i: (i, 0, 0)),
            pl.BlockSpec((PV, PV), lambda i: (0, 0)),
        ],
        out_specs=[
            pl.BlockSpec((1, TB, 1), lambda i: (i, 0, 0)),
            pl.BlockSpec((1, 1), lambda i: (0, 0)),
        ],
        out_shape=[
            jax.ShapeDtypeStruct((NSTEPS, TB, 1), jnp.int32),
            jax.ShapeDtypeStruct((1, 1), jnp.float32),
        ],
        scratch_shapes=[
            pltpu.VMEM((PV, 1), jnp.float32),
            pltpu.SMEM((1, 1), jnp.float32),
        ],
    )(mb, yv, tab)

    return yp.reshape(NTOK), loss[0, 0]


# speedup vs baseline: 1.3532x; 1.3532x over previous
import functools

import jax
import jax.numpy as jnp
import numpy as np
from jax.experimental import pallas as pl
from jax.experimental.pallas import tpu as pltpu

VOCAB = 1000
PV = 1024
NTOK = 65536
TB = 256
NSTEPS = NTOK // TB

_TINY = np.float32(np.finfo(np.float32).tiny)
_K0 = 0
_K1 = 42
_KS2 = np.uint32(_K0 ^ _K1 ^ 0x1BD11BDA)
_ROT = ((13, 15, 26, 6), (17, 29, 16, 24))


def _rotl(x, r):
    return (x << jnp.uint32(r)) | (x >> jnp.uint32(32 - r))


def _threefry_bits(cnt_lo):
    ks = (jnp.uint32(_K0), jnp.uint32(_K1), _KS2)
    x0 = jnp.full_like(cnt_lo, ks[0])
    x1 = cnt_lo + ks[1]
    for i in range(5):
        for r in _ROT[i % 2]:
            x0 = x0 + x1
            x1 = _rotl(x1, r)
            x1 = x1 ^ x0
        x0 = x0 + ks[(i + 1) % 3]
        x1 = x1 + ks[(i + 2) % 3] + jnp.uint32(i + 1)
    return x0 ^ x1


def _body(mb_ref, y_ref, tabhi_ref, tablo_ref, yp_ref, loss_ref, acc_ref):
    step = pl.program_id(0)
    col = jax.lax.broadcasted_iota(jnp.int32, (TB, PV), 1)
    valid = col < VOCAB

    @pl.when(step == 0)
    def _init():
        acc_ref[0, 0] = 0.0

    mb = mb_ref[0]
    yb = y_ref[0]

    onehot = (col == mb).astype(jnp.bfloat16)
    logits = (jnp.dot(onehot, tabhi_ref[...],
                      preferred_element_type=jnp.float32)
              + jnp.dot(onehot, tablo_ref[...],
                        preferred_element_type=jnp.float32))

    row = jax.lax.broadcasted_iota(jnp.uint32, (TB, PV), 0)
    cnt = (jnp.uint32(step * (TB * VOCAB))
           + (row * jnp.uint32(VOCAB) + col.astype(jnp.uint32)))
    bits = _threefry_bits(cnt)

    fb = (bits >> jnp.uint32(9)) | jnp.uint32(0x3F800000)
    f = jax.lax.bitcast_convert_type(fb, jnp.float32) - jnp.float32(1.0)
    u = jnp.maximum(_TINY, f + _TINY)
    g = -jnp.log(-jnp.log(u))

    z = jnp.where(valid, logits + g, -jnp.inf)
    zmax = jnp.max(z, axis=1, keepdims=True)
    win = jnp.min(jnp.where(z == zmax, col, PV), axis=1, keepdims=True)
    yp_ref[0] = win

    tgt = jnp.sum(jnp.where(col == yb, logits, 0.0), axis=1, keepdims=True)
    lmax = jnp.max(jnp.where(valid, logits, -jnp.inf), axis=1, keepdims=True)
    s = jnp.sum(jnp.where(valid, jnp.exp(logits - lmax), 0.0), axis=1,
                keepdims=True)
    lse_tok = lmax + jnp.log(s)
    acc_ref[0, 0] += jnp.sum(lse_tok - tgt)
    loss_ref[...] = jnp.full((1, 1), acc_ref[0, 0] * jnp.float32(1.0 / NTOK),
                             jnp.float32)


@jax.jit
def kernel(mini_batch, y, embed_weight):
    mb = mini_batch.reshape(NSTEPS, TB, 1)
    yv = y.reshape(NSTEPS, TB, 1)
    tab = jnp.pad(embed_weight,
                  ((0, PV - VOCAB), (0, PV - VOCAB)))
    tab_hi = tab.astype(jnp.bfloat16)
    tab_lo = (tab - tab_hi.astype(jnp.float32)).astype(jnp.bfloat16)

    yp, loss = pl.pallas_call(
        _body,
        grid=(NSTEPS,),
        in_specs=[
            pl.BlockSpec((1, TB, 1), lambda i: (i, 0, 0)),
            pl.BlockSpec((1, TB, 1), lambda i: (i, 0, 0)),
            pl.BlockSpec((PV, PV), lambda i: (0, 0)),
            pl.BlockSpec((PV, PV), lambda i: (0, 0)),
        ],
        out_specs=[
            pl.BlockSpec((1, TB, 1), lambda i: (i, 0, 0)),
            pl.BlockSpec((1, 1), lambda i: (0, 0)),
        ],
        out_shape=[
            jax.ShapeDtypeStruct((NSTEPS, TB, 1), jnp.int32),
            jax.ShapeDtypeStruct((1, 1), jnp.float32),
        ],
        scratch_shapes=[
            pltpu.SMEM((1, 1), jnp.float32),
        ],
    )(mb, yv, tab_hi, tab_lo)

    return yp.reshape(NTOK), loss[0, 0]


# speedup vs baseline: 1.3889x; 1.0264x over previous
import functools

import jax
import jax.numpy as jnp
import numpy as np
from jax.experimental import pallas as pl
from jax.experimental.pallas import tpu as pltpu

VOCAB = 1000
PV = 1024
NTOK = 65536
TB = 512
NSTEPS = NTOK // TB

_TINY = np.float32(np.finfo(np.float32).tiny)
_K0 = 0
_K1 = 42
_KS2 = np.uint32(_K0 ^ _K1 ^ 0x1BD11BDA)
_ROT = ((13, 15, 26, 6), (17, 29, 16, 24))


def _rotl(x, r):
    return (x << jnp.uint32(r)) | (x >> jnp.uint32(32 - r))


def _threefry_bits(cnt_lo):
    ks = (jnp.uint32(_K0), jnp.uint32(_K1), _KS2)
    x0 = jnp.full_like(cnt_lo, ks[0])
    x1 = cnt_lo + ks[1]
    for i in range(5):
        for r in _ROT[i % 2]:
            x0 = x0 + x1
            x1 = _rotl(x1, r)
            x1 = x1 ^ x0
        x0 = x0 + ks[(i + 1) % 3]
        x1 = x1 + ks[(i + 2) % 3] + jnp.uint32(i + 1)
    return x0 ^ x1


def _body(mb_ref, y_ref, tabhi_ref, tablo_ref, yp_ref, loss_ref, acc_ref):
    step = pl.program_id(0)
    col = jax.lax.broadcasted_iota(jnp.int32, (TB, PV), 1)
    valid = col < VOCAB

    @pl.when(step == 0)
    def _init():
        acc_ref[0, 0] = 0.0

    mb = mb_ref[0]
    yb = y_ref[0]

    onehot = (col == mb).astype(jnp.bfloat16)
    logits = (jnp.dot(onehot, tabhi_ref[...],
                      preferred_element_type=jnp.float32)
              + jnp.dot(onehot, tablo_ref[...],
                        preferred_element_type=jnp.float32))

    row = jax.lax.broadcasted_iota(jnp.uint32, (TB, PV), 0)
    cnt = (jnp.uint32(step * (TB * VOCAB))
           + (row * jnp.uint32(VOCAB) + col.astype(jnp.uint32)))
    bits = _threefry_bits(cnt)

    fb = (bits >> jnp.uint32(9)) | jnp.uint32(0x3F800000)
    f = jax.lax.bitcast_convert_type(fb, jnp.float32) - jnp.float32(1.0)
    u = jnp.maximum(_TINY, f + _TINY)
    g = -jnp.log(-jnp.log(u))

    z = jnp.where(valid, logits + g, -jnp.inf)
    zmax = jnp.max(z, axis=1, keepdims=True)
    win = jnp.min(jnp.where(z == zmax, col, PV), axis=1, keepdims=True)
    yp_ref[0] = win

    tgt = jnp.sum(jnp.where(col == yb, logits, 0.0), axis=1, keepdims=True)
    lmax = jnp.max(jnp.where(valid, logits, -jnp.inf), axis=1, keepdims=True)
    s = jnp.sum(jnp.where(valid, jnp.exp(logits - lmax), 0.0), axis=1,
                keepdims=True)
    lse_tok = lmax + jnp.log(s)
    acc_ref[0, 0] += jnp.sum(lse_tok - tgt)
    loss_ref[...] = jnp.full((1, 1), acc_ref[0, 0] * jnp.float32(1.0 / NTOK),
                             jnp.float32)


@jax.jit
def kernel(mini_batch, y, embed_weight):
    mb = mini_batch.reshape(NSTEPS, TB, 1)
    yv = y.reshape(NSTEPS, TB, 1)
    tab = jnp.pad(embed_weight,
                  ((0, PV - VOCAB), (0, PV - VOCAB)))
    tab_hi = tab.astype(jnp.bfloat16)
    tab_lo = (tab - tab_hi.astype(jnp.float32)).astype(jnp.bfloat16)

    yp, loss = pl.pallas_call(
        _body,
        grid=(NSTEPS,),
        in_specs=[
            pl.BlockSpec((1, TB, 1), lambda i: (i, 0, 0)),
            pl.BlockSpec((1, TB, 1), lambda i: (i, 0, 0)),
            pl.BlockSpec((PV, PV), lambda i: (0, 0)),
            pl.BlockSpec((PV, PV), lambda i: (0, 0)),
        ],
        out_specs=[
            pl.BlockSpec((1, TB, 1), lambda i: (i, 0, 0)),
            pl.BlockSpec((1, 1), lambda i: (0, 0)),
        ],
        out_shape=[
            jax.ShapeDtypeStruct((NSTEPS, TB, 1), jnp.int32),
            jax.ShapeDtypeStruct((1, 1), jnp.float32),
        ],
        scratch_shapes=[
            pltpu.SMEM((1, 1), jnp.float32),
        ],
    )(mb, yv, tab_hi, tab_lo)

    return yp.reshape(NTOK), loss[0, 0]


# speedup vs baseline: 1.4300x; 1.0296x over previous
import functools

import jax
import jax.numpy as jnp
import numpy as np
from jax.experimental import pallas as pl
from jax.experimental.pallas import tpu as pltpu

VOCAB = 1000
PV = 1024
NTOK = 65536
TB = 512
NSTEPS = NTOK // TB

_TINY = np.float32(np.finfo(np.float32).tiny)
_K0 = 0
_K1 = 42
_KS2 = np.uint32(_K0 ^ _K1 ^ 0x1BD11BDA)
_ROT = ((13, 15, 26, 6), (17, 29, 16, 24))


def _rotl(x, r):
    return (x << jnp.uint32(r)) | (x >> jnp.uint32(32 - r))


def _threefry_bits(x1):
    ks = (np.uint32(_K0), np.uint32(_K1), _KS2)
    x0 = x1
    x1 = _rotl(x1, _ROT[0][0]) ^ x0
    for r in _ROT[0][1:]:
        x0 = x0 + x1
        x1 = _rotl(x1, r)
        x1 = x1 ^ x0
    x0 = x0 + ks[1]
    x1 = x1 + np.uint32(ks[2] + np.uint32(1))
    for i in range(1, 5):
        for r in _ROT[i % 2]:
            x0 = x0 + x1
            x1 = _rotl(x1, r)
            x1 = x1 ^ x0
        x0 = x0 + ks[(i + 1) % 3]
        x1 = x1 + np.uint32(ks[(i + 2) % 3] + np.uint32(i + 1))
    return x0 ^ x1


def _body(mb_ref, y_ref, tabhi_ref, tablo_ref, yp_ref, loss_ref, acc_ref,
          cnt_ref):
    step = pl.program_id(0)
    col = jax.lax.broadcasted_iota(jnp.int32, (TB, PV), 1)
    valid = col < VOCAB

    @pl.when(step == 0)
    def _init():
        acc_ref[0, 0] = 0.0
        rowi = jax.lax.broadcasted_iota(jnp.uint32, (TB, PV), 0)
        cnt_ref[...] = (rowi * jnp.uint32(VOCAB) + col.astype(jnp.uint32)
                        + jnp.uint32(_K1))

    mb = mb_ref[0]
    yb = y_ref[0]

    onehot = (col == mb).astype(jnp.bfloat16)
    logits = (jnp.dot(onehot, tabhi_ref[...],
                      preferred_element_type=jnp.float32)
              + jnp.dot(onehot, tablo_ref[...],
                        preferred_element_type=jnp.float32))

    bits = _threefry_bits(cnt_ref[...]
                          + (step * (TB * VOCAB)).astype(jnp.uint32))

    fb = (bits >> jnp.uint32(9)) | jnp.uint32(0x3F800000)
    f = jax.lax.bitcast_convert_type(fb, jnp.float32) - jnp.float32(1.0)
    u = jnp.maximum(_TINY, f + _TINY)
    g = -jnp.log(-jnp.log(u))

    z = jnp.where(valid, logits + g, -jnp.inf)
    zmax = jnp.max(z, axis=1, keepdims=True)
    win = jnp.min(jnp.where(z == zmax, col, PV), axis=1, keepdims=True)
    yp_ref[0] = win

    tgt = jnp.sum(jnp.where(col == yb, logits, 0.0), axis=1, keepdims=True)
    lmax = jnp.max(jnp.where(valid, logits, -jnp.inf), axis=1, keepdims=True)
    s = jnp.sum(jnp.where(valid, jnp.exp(logits - lmax), 0.0), axis=1,
                keepdims=True)
    lse_tok = lmax + jnp.log(s)
    acc_ref[0, 0] += jnp.sum(lse_tok - tgt)
    loss_ref[...] = jnp.full((1, 1), acc_ref[0, 0] * jnp.float32(1.0 / NTOK),
                             jnp.float32)


@jax.jit
def kernel(mini_batch, y, embed_weight):
    mb = mini_batch.reshape(NSTEPS, TB, 1)
    yv = y.reshape(NSTEPS, TB, 1)
    tab = jnp.pad(embed_weight,
                  ((0, PV - VOCAB), (0, PV - VOCAB)))
    tab_hi = tab.astype(jnp.bfloat16)
    tab_lo = (tab - tab_hi.astype(jnp.float32)).astype(jnp.bfloat16)

    yp, loss = pl.pallas_call(
        _body,
        grid=(NSTEPS,),
        in_specs=[
            pl.BlockSpec((1, TB, 1), lambda i: (i, 0, 0)),
            pl.BlockSpec((1, TB, 1), lambda i: (i, 0, 0)),
            pl.BlockSpec((PV, PV), lambda i: (0, 0)),
            pl.BlockSpec((PV, PV), lambda i: (0, 0)),
        ],
        out_specs=[
            pl.BlockSpec((1, TB, 1), lambda i: (i, 0, 0)),
            pl.BlockSpec((1, 1), lambda i: (0, 0)),
        ],
        out_shape=[
            jax.ShapeDtypeStruct((NSTEPS, TB, 1), jnp.int32),
            jax.ShapeDtypeStruct((1, 1), jnp.float32),
        ],
        scratch_shapes=[
            pltpu.SMEM((1, 1), jnp.float32),
            pltpu.VMEM((TB, PV), jnp.uint32),
        ],
    )(mb, yv, tab_hi, tab_lo)

    return yp.reshape(NTOK), loss[0, 0]


# speedup vs baseline: 1.5071x; 1.0539x over previous
import functools

import jax
import jax.numpy as jnp
import numpy as np
from jax.experimental import pallas as pl
from jax.experimental.pallas import tpu as pltpu
from jax.experimental.pallas import tpu_sc as plsc

VOCAB = 1000
PV = 1024
NTOK = 65536
TB = 512
NSTEPS = NTOK // TB

_TINY = np.float32(np.finfo(np.float32).tiny)
_K0 = 0
_K1 = 42
_KS2 = np.uint32(_K0 ^ _K1 ^ 0x1BD11BDA)
_ROT = ((13, 15, 26, 6), (17, 29, 16, 24))


def _rotl(x, r):
    return (x << jnp.uint32(r)) | (x >> jnp.uint32(32 - r))


def _threefry_bits(x1):
    ks = (np.uint32(_K0), np.uint32(_K1), _KS2)
    x0 = x1
    x1 = _rotl(x1, _ROT[0][0]) ^ x0
    for r in _ROT[0][1:]:
        x0 = x0 + x1
        x1 = _rotl(x1, r)
        x1 = x1 ^ x0
    x0 = x0 + ks[1]
    x1 = x1 + np.uint32(ks[2] + np.uint32(1))
    for i in range(1, 5):
        for r in _ROT[i % 2]:
            x0 = x0 + x1
            x1 = _rotl(x1, r)
            x1 = x1 ^ x0
        x0 = x0 + ks[(i + 1) % 3]
        x1 = x1 + np.uint32(ks[(i + 2) % 3] + np.uint32(i + 1))
    return x0 ^ x1


def _body(mb_ref, tabhi_ref, tablo_ref, yp_ref, loss_ref, acc_ref,
          cnt_ref):
    step = pl.program_id(0)
    col = jax.lax.broadcasted_iota(jnp.int32, (TB, PV), 1)
    valid = col < VOCAB

    @pl.when(step == 0)
    def _init():
        acc_ref[0, 0] = 0.0
        rowi = jax.lax.broadcasted_iota(jnp.uint32, (TB, PV), 0)
        cnt_ref[...] = (rowi * jnp.uint32(VOCAB) + col.astype(jnp.uint32)
                        + jnp.uint32(_K1))

    mb = mb_ref[0]

    onehot = (col == mb).astype(jnp.bfloat16)
    logits = (jnp.dot(onehot, tabhi_ref[...],
                      preferred_element_type=jnp.float32)
              + jnp.dot(onehot, tablo_ref[...],
                        preferred_element_type=jnp.float32))

    bits = _threefry_bits(cnt_ref[...]
                          + (step * (TB * VOCAB)).astype(jnp.uint32))

    fb = (bits >> jnp.uint32(9)) | jnp.uint32(0x3F800000)
    f = jax.lax.bitcast_convert_type(fb, jnp.float32) - jnp.float32(1.0)
    u = jnp.maximum(_TINY, f + _TINY)
    g = -jnp.log(-jnp.log(u))

    logits_m = jnp.where(valid, logits, -jnp.inf)
    z = logits_m + g
    zmax = jnp.max(z, axis=1, keepdims=True)
    win = jnp.min(jnp.where(z == zmax, col, PV), axis=1, keepdims=True)
    yp_ref[0] = win

    lmax = jnp.max(logits_m, axis=1, keepdims=True)
    s = jnp.sum(jnp.exp(logits_m - lmax), axis=1, keepdims=True)
    lse_tok = lmax + jnp.log(s)
    acc_ref[0, 0] += jnp.sum(lse_tok)
    loss_ref[...] = jnp.full((1, 1), acc_ref[0, 0], jnp.float32)


_NC, _NS, _NL = 2, 16, 16
_NW = _NC * _NS
_BPW = NTOK // _NW
_CH = _BPW // _NL


def _tgt_body(mb_hbm, y_hbm, tabf_hbm, out_hbm, mb_v, y_v, idx_v, val_v,
              acc_v, sem):
    wid = jax.lax.axis_index("s") * _NC + jax.lax.axis_index("c")
    base = wid * _BPW
    pltpu.sync_copy(mb_hbm.at[pl.ds(base, _BPW)], mb_v)
    pltpu.sync_copy(y_hbm.at[pl.ds(base, _BPW)], y_v)

    def _mk_idx(i, carry):
        sl = pl.ds(i * _NL, _NL)
        idx_v[sl] = mb_v[sl] * VOCAB + y_v[sl]
        return carry

    jax.lax.fori_loop(0, _CH, _mk_idx, 0)
    pltpu.async_copy(tabf_hbm.at[idx_v], val_v, sem).wait()

    def _acc(i, acc):
        return acc + val_v[pl.ds(i * _NL, _NL)]

    acc_v[...] = jax.lax.fori_loop(0, _CH, _acc,
                                   jnp.zeros((_NL,), jnp.float32))
    pltpu.sync_copy(acc_v, out_hbm.at[wid])


_tgt_partials = functools.partial(
    pl.kernel,
    _tgt_body,
    out_type=jax.ShapeDtypeStruct((_NW, _NL), jnp.float32),
    mesh=plsc.VectorSubcoreMesh(core_axis_name="c", subcore_axis_name="s"),
    scratch_types=[
        pltpu.VMEM((_BPW,), jnp.int32),
        pltpu.VMEM((_BPW,), jnp.int32),
        pltpu.VMEM((_BPW,), jnp.int32),
        pltpu.VMEM((_BPW,), jnp.float32),
        pltpu.VMEM((_NL,), jnp.float32),
        pltpu.SemaphoreType.DMA,
    ],
)()


@jax.jit
def kernel(mini_batch, y, embed_weight):
    mb = mini_batch.reshape(NSTEPS, TB, 1)
    tab = jnp.pad(embed_weight,
                  ((0, PV - VOCAB), (0, PV - VOCAB)))
    tab_hi = tab.astype(jnp.bfloat16)
    tab_lo = (tab - tab_hi.astype(jnp.float32)).astype(jnp.bfloat16)

    yp, lse_sum = pl.pallas_call(
        _body,
        grid=(NSTEPS,),
        in_specs=[
            pl.BlockSpec((1, TB, 1), lambda i: (i, 0, 0)),
            pl.BlockSpec((PV, PV), lambda i: (0, 0)),
            pl.BlockSpec((PV, PV), lambda i: (0, 0)),
        ],
        out_specs=[
            pl.BlockSpec((1, TB, 1), lambda i: (i, 0, 0)),
            pl.BlockSpec((1, 1), lambda i: (0, 0)),
        ],
        out_shape=[
            jax.ShapeDtypeStruct((NSTEPS, TB, 1), jnp.int32),
            jax.ShapeDtypeStruct((1, 1), jnp.float32),
        ],
        scratch_shapes=[
            pltpu.SMEM((1, 1), jnp.float32),
            pltpu.VMEM((TB, PV), jnp.uint32),
        ],
    )(mb, tab_hi, tab_lo)

    tgt_part = _tgt_partials(mini_batch.reshape(NTOK), y.reshape(NTOK),
                             embed_weight.reshape(VOCAB * VOCAB))
    loss = (lse_sum[0, 0] - jnp.sum(tgt_part)) * jnp.float32(1.0 / NTOK)
    return yp.reshape(NTOK), loss
